# baseline (speedup 1.0000x reference)
import functools
import math

import jax
import jax.numpy as jnp
from jax import lax
from jax.experimental import pallas as pl
from jax.experimental.pallas import tpu as pltpu
from jax.experimental.pallas import tpu_sc as plsc

NHEAD = 12
MAX_PATH_LEN = 4
MAX_VISITS = 2


def _sc_gather(table, idx):
    V, D = table.shape
    (N,) = idx.shape
    info = plsc.get_sparse_core_info()
    NW = info.num_cores * info.num_subcores
    b_per_w = N // NW
    mesh = plsc.VectorSubcoreMesh(core_axis_name="c", subcore_axis_name="s")

    @functools.partial(
        pl.kernel,
        mesh=mesh,
        out_type=jax.ShapeDtypeStruct((N, D), jnp.float32),
        scratch_types=[
            pltpu.VMEM((b_per_w,), jnp.int32),
            pltpu.VMEM((b_per_w, D), jnp.float32),
            pltpu.SemaphoreType.DMA,
        ],
    )
    def k(table_hbm, idx_hbm, out_hbm, idx_v, rows_v, sem):
        wid = lax.axis_index("s") * info.num_cores + lax.axis_index("c")
        base = wid * b_per_w
        pltpu.sync_copy(idx_hbm.at[pl.ds(base, b_per_w)], idx_v)
        pltpu.async_copy(table_hbm.at[idx_v], rows_v, sem).wait()
        pltpu.sync_copy(rows_v, out_hbm.at[pl.ds(base, b_per_w)])

    return k(table, idx)


def _router_kernel(x_ref, act_ref, vis_ref, w1_ref, b1_ref, w2_ref, b2_ref,
                   eidx_ref, route_ref, vis_out_ref, ent_ref, acc_ref,
                   *, nblk, S, E):
    j = pl.program_id(0)

    @pl.when(j == 0)
    def _():
        acc_ref[...] = jnp.zeros_like(acc_ref)

    acc_ref[...] += jnp.sum(x_ref[...], axis=1)

    @pl.when(j == nblk - 1)
    def _():
        summary = acc_ref[...] / float(S)
        h = jnp.maximum(
            lax.dot_general(summary, w1_ref[...], (((1,), (1,)), ((), ())),
                            preferred_element_type=jnp.float32)
            + b1_ref[...][None, :], 0.0)
        logits = lax.dot_general(h, w2_ref[...], (((1,), (1,)), ((), ())),
                                 preferred_element_type=jnp.float32) \
            + b2_ref[...][None, :]
        B = logits.shape[0]
        vis = vis_ref[...][:, :E]
        masked = jnp.where(vis >= MAX_VISITS, -1e9, logits[:, :E])
        full = jnp.concatenate([masked, logits[:, E:E + 1]], axis=1)
        mx = jnp.max(full, axis=1, keepdims=True)
        ex = jnp.exp(full - mx)
        probs = ex / jnp.sum(ex, axis=1, keepdims=True)
        safe = jnp.maximum(probs, 1e-9)
        ent = -jnp.sum(safe * jnp.log(safe), axis=1)
        active = act_ref[...][:, 0]
        n_act = jnp.sum(active.astype(jnp.float32))
        step_ent = jnp.where(
            n_act > 0.0,
            jnp.sum(ent * active.astype(jnp.float32)) / jnp.maximum(n_act, 1.0),
            0.0)
        ci = lax.broadcasted_iota(jnp.int32, full.shape, 1)
        ismax = full >= jnp.max(full, axis=1, keepdims=True)
        choice = jnp.min(jnp.where(ismax, ci, E + 1), axis=1)
        route = ((active == 1) & (choice < E)).astype(jnp.int32)
        eidx = jnp.where(route == 1, choice, 0)
        lanes = lax.broadcasted_iota(jnp.int32, vis_ref.shape, 1)
        onehot = ((lanes == eidx[:, None]) & (lanes < E)).astype(jnp.int32)
        vis_out_ref[...] = vis_ref[...] + onehot * route[:, None]
        eidx_ref[...] = jnp.broadcast_to(eidx[:, None], eidx_ref.shape)
        route_ref[...] = jnp.broadcast_to(route[:, None], route_ref.shape)
        ent_ref[...] = jnp.full(ent_ref.shape, step_ent, jnp.float32)


def _router_step(x, act, vis, r_w1, r_b1, r_w2, r_b2):
    B, S, d = x.shape
    E = r_w2.shape[0] - 1
    SBLK = 512
    nblk = S // SBLK
    out = pl.pallas_call(
        functools.partial(_router_kernel, nblk=nblk, S=S, E=E),
        grid=(nblk,),
        in_specs=[
            pl.BlockSpec((B, SBLK, d), lambda j: (0, j, 0)),
            pl.BlockSpec((B, 128), lambda j: (0, 0)),
            pl.BlockSpec((B, 128), lambda j: (0, 0)),
            pl.BlockSpec(r_w1.shape, lambda j: (0, 0)),
            pl.BlockSpec(r_b1.shape, lambda j: (0,)),
            pl.BlockSpec(r_w2.shape, lambda j: (0, 0)),
            pl.BlockSpec(r_b2.shape, lambda j: (0,)),
        ],
        out_specs=[
            pl.BlockSpec((B, 128), lambda j: (0, 0)),
            pl.BlockSpec((B, 128), lambda j: (0, 0)),
            pl.BlockSpec((B, 128), lambda j: (0, 0)),
            pl.BlockSpec((B, 128), lambda j: (0, 0)),
        ],
        out_shape=[
            jax.ShapeDtypeStruct((B, 128), jnp.int32),
            jax.ShapeDtypeStruct((B, 128), jnp.int32),
            jax.ShapeDtypeStruct((B, 128), jnp.int32),
            jax.ShapeDtypeStruct((B, 128), jnp.float32),
        ],
        scratch_shapes=[pltpu.VMEM((B, d), jnp.float32)],
    )(x, act, vis, r_w1, r_b1, r_w2, r_b2)
    return out


def _attn_kernel(eidx_ref, route_ref, x_ref, w_ref, b_ref, out_ref,
                 k_scr, v_scr, *, H, dh, d, S, BQ):
    b = pl.program_id(0)
    ph = pl.program_id(1)
    qi = pl.program_id(2)

    @pl.when(route_ref[b] == 1)
    def _():
        scale = 1.0 / math.sqrt(dh)
        w = w_ref[0]
        bias = b_ref[0]
        xb = x_ref[0]

        @pl.when(ph == 0)
        def _():
            kp = lax.dot_general(
                xb, w[d:2 * d, :], (((1,), (1,)), ((), ())),
                preferred_element_type=jnp.float32) + bias[:, d:2 * d]
            vp = lax.dot_general(
                xb, w[2 * d:3 * d, :], (((1,), (1,)), ((), ())),
                preferred_element_type=jnp.float32) + bias[:, 2 * d:3 * d]
            k_scr[pl.ds(qi * BQ, BQ), :] = kp
            one = jnp.ones((BQ, 1), jnp.float32)
            z = jnp.zeros((BQ, dh - 1), jnp.float32)
            for h in range(H):
                sl = slice(h * dh, (h + 1) * dh)
                v_scr[h, pl.ds(qi * BQ, BQ), :] = jnp.concatenate(
                    [vp[:, sl], one, z], axis=1)

        @pl.when(ph == 1)
        def _():
            q = lax.dot_general(xb, w[:d, :], (((1,), (1,)), ((), ())),
                                preferred_element_type=jnp.float32) \
                + bias[:, :d]
            outs = []
            for h in range(H):
                sl = slice(h * dh, (h + 1) * dh)
                s = lax.dot_general(q[:, sl] * scale, k_scr[:, sl],
                                    (((1,), (1,)), ((), ())),
                                    preferred_element_type=jnp.float32)
                p = jnp.exp(s)
                ol = lax.dot_general(p, v_scr[h], (((1,), (0,)), ((), ())),
                                     preferred_element_type=jnp.float32)
                outs.append(ol[:, :dh] / ol[:, dh:dh + 1])
            out_ref[...] = jnp.concatenate(outs, axis=1)[None]


def _attention(x, Wqkv, bqkv, eidx, route):
    B, S, d = x.shape
    E, d3, _ = Wqkv.shape
    H, dh = NHEAD, d // NHEAD
    BQ = 512
    grid = (B, 2, S // BQ)
    return pl.pallas_call(
        functools.partial(_attn_kernel, H=H, dh=dh, d=d, S=S, BQ=BQ),
        grid_spec=pltpu.PrefetchScalarGridSpec(
            num_scalar_prefetch=2,
            grid=grid,
            in_specs=[
                pl.BlockSpec((1, BQ, d), lambda b, ph, qi, e, r: (b, qi, 0)),
                pl.BlockSpec((1, d3, d), lambda b, ph, qi, e, r: (e[b], 0, 0)),
                pl.BlockSpec((1, 1, d3), lambda b, ph, qi, e, r: (e[b], 0, 0)),
            ],
            out_specs=pl.BlockSpec(
                (1, BQ, d),
                lambda b, ph, qi, e, r: (b, jnp.where(ph == 1, qi, 0), 0)),
            scratch_shapes=[
                pltpu.VMEM((S, d), jnp.float32),
                pltpu.VMEM((H, S, 2 * dh), jnp.float32),
            ],
        ),
        out_shape=jax.ShapeDtypeStruct((B, S, d), jnp.float32),
        compiler_params=pltpu.CompilerParams(
            vmem_limit_bytes=62 * 1024 * 1024),
    )(eidx, route, x, Wqkv, bqkv[:, None, :])


def _ln(x, g, b):
    m = jnp.mean(x, axis=-1, keepdims=True)
    v = jnp.mean((x - m) ** 2, axis=-1, keepdims=True)
    return (x - m) / jnp.sqrt(v + 1e-5) * g + b


def _mlp_kernel(eidx_ref, route_ref, x_ref, o_ref, wo_ref, bo_ref,
                g1_ref, b1n_ref, w1_ref, b1f_ref, w2_ref, b2f_ref,
                g2_ref, b2n_ref, tag_ref, out_ref):
    b = pl.program_id(0)

    @pl.when(route_ref[b] == 1)
    def _():
        o = lax.dot_general(o_ref[0], wo_ref[0], (((1,), (1,)), ((), ())),
                            preferred_element_type=jnp.float32) + bo_ref[0]
        x1 = _ln(x_ref[0] + o, g1_ref[0], b1n_ref[0])
        f = jnp.maximum(
            lax.dot_general(x1, w1_ref[0], (((1,), (1,)), ((), ())),
                            preferred_element_type=jnp.float32)
            + b1f_ref[0], 0.0)
        y = lax.dot_general(f, w2_ref[0], (((1,), (1,)), ((), ())),
                            preferred_element_type=jnp.float32) + b2f_ref[0]
        out_ref[...] = (_ln(x1 + y, g2_ref[0], b2n_ref[0]) + tag_ref[0])[None]

    @pl.when(route_ref[b] == 0)
    def _():
        out_ref[...] = x_ref[...]


def _mlp(x, o, Wo, bo, g1, b1n, W1, b1f, W2, b2f, g2, b2n, tag, eidx, route):
    B, S, d = x.shape
    E, ff, _ = W1.shape
    MB = 512
    grid = (B, S // MB)
    return pl.pallas_call(
        _mlp_kernel,
        grid_spec=pltpu.PrefetchScalarGridSpec(
            num_scalar_prefetch=2,
            grid=grid,
            in_specs=[
                pl.BlockSpec((1, MB, d), lambda b, m, e, r: (b, m, 0)),
                pl.BlockSpec((1, MB, d), lambda b, m, e, r: (b, m, 0)),
                pl.BlockSpec((1, d, d), lambda b, m, e, r: (e[b], 0, 0)),
                pl.BlockSpec((1, 1, d), lambda b, m, e, r: (e[b], 0, 0)),
                pl.BlockSpec((1, 1, d), lambda b, m, e, r: (e[b], 0, 0)),
                pl.BlockSpec((1, 1, d), lambda b, m, e, r: (e[b], 0, 0)),
                pl.BlockSpec((1, ff, d), lambda b, m, e, r: (e[b], 0, 0)),
                pl.BlockSpec((1, 1, ff), lambda b, m, e, r: (e[b], 0, 0)),
                pl.BlockSpec((1, d, ff), lambda b, m, e, r: (e[b], 0, 0)),
                pl.BlockSpec((1, 1, d), lambda b, m, e, r: (e[b], 0, 0)),
                pl.BlockSpec((1, 1, d), lambda b, m, e, r: (e[b], 0, 0)),
                pl.BlockSpec((1, 1, d), lambda b, m, e, r: (e[b], 0, 0)),
                pl.BlockSpec((1, 1, d), lambda b, m, e, r: (e[b], 0, 0)),
            ],
            out_specs=pl.BlockSpec((1, MB, d), lambda b, m, e, r: (b, m, 0)),
        ),
        out_shape=jax.ShapeDtypeStruct((B, S, d), jnp.float32),
    )(eidx, route, x, o, Wo, bo[:, None, :], g1[:, None, :], b1n[:, None, :],
      W1, b1f[:, None, :], W2, b2f[:, None, :], g2[:, None, :],
      b2n[:, None, :], tag[:, None, :])


def _lm_kernel(x_ref, w_ref, b_ref, out_ref):
    out_ref[...] = (
        lax.dot_general(x_ref[0].astype(jnp.bfloat16),
                        w_ref[...].astype(jnp.bfloat16),
                        (((1,), (1,)), ((), ())),
                        preferred_element_type=jnp.float32)
        + b_ref[...][None, :])[None]


def _lm_head(x, lm_w, lm_b):
    B, S, d = x.shape
    V = lm_w.shape[0]
    NB = 1024
    grid = (B, V // NB)
    return pl.pallas_call(
        _lm_kernel,
        grid=grid,
        in_specs=[
            pl.BlockSpec((1, S, d), lambda b, n: (b, 0, 0)),
            pl.BlockSpec((NB, d), lambda b, n: (n, 0)),
            pl.BlockSpec((NB,), lambda b, n: (n,)),
        ],
        out_specs=pl.BlockSpec((1, S, NB), lambda b, n: (b, 0, n)),
        out_shape=jax.ShapeDtypeStruct((B, S, V), jnp.float32),
    )(x, lm_w, lm_b)


def kernel(input_ids_seq, emb, Wqkv, bqkv, Wo, bo, ln1_g, ln1_b, W1, b1,
           W2, b2, ln2_g, ln2_b, tag, r_w1, r_b1, r_w2, r_b2, lm_w, lm_b):
    B, S = input_ids_seq.shape
    V, d = emb.shape
    E = Wqkv.shape[0]

    pos = jnp.arange(S, dtype=jnp.float32)[:, None]
    div = jnp.exp(jnp.arange(0, d, 2, dtype=jnp.float32)
                  * (-math.log(10000.0) / d))
    pe = jnp.zeros((S, d), jnp.float32)
    pe = pe.at[:, 0::2].set(jnp.sin(pos * div)).at[:, 1::2].set(jnp.cos(pos * div))

    rows = _sc_gather(emb, input_ids_seq.reshape(-1))
    x = rows.reshape(B, S, d) * math.sqrt(d) + pe[None, :, :]

    act = jnp.ones((B, 128), jnp.int32)
    vis = jnp.zeros((B, 128), jnp.int32)
    total_ent = jnp.float32(0.0)
    for _ in range(MAX_PATH_LEN):
        eidx_a, route_a, vis, ent_a = _router_step(
            x, act, vis, r_w1, r_b1, r_w2, r_b2)
        eidx = eidx_a[:, 0]
        route = route_a[:, 0]
        total_ent = total_ent + ent_a[0, 0]
        o = _attention(x, Wqkv, bqkv, eidx, route)
        x = _mlp(x, o, Wo, bo, ln1_g, ln1_b, W1, b1, W2, b2,
                 ln2_g, ln2_b, tag, eidx, route)
        act = route_a
    lm_logits = _lm_head(x, lm_w, lm_b)
    return lm_logits, total_ent
